# Initial kernel scaffold; baseline (speedup 1.0000x reference)
import jax
import jax.numpy as jnp
from jax.experimental import pallas as pl


def kernel(xyz, params):
    raise NotImplementedError("write your pallas kernel here")


# speedup vs baseline: 1.0004x; 1.0004x over previous
import jax
import jax.numpy as jnp
import numpy as np
from jax.experimental import pallas as pl


def _fps_single(xyz, K):
    N = xyz.shape[0]
    def body(i, state):
        idxs, dists = state
        last = xyz[idxs[i - 1]]
        d = jnp.sum((xyz - last) ** 2, axis=-1)
        dists = jnp.minimum(dists, d)
        idxs = idxs.at[i].set(jnp.argmax(dists).astype(jnp.int32))
        return (idxs, dists)
    idxs0 = jnp.zeros((K,), jnp.int32)
    dists0 = jnp.full((N,), jnp.inf, dtype=jnp.float32)
    idxs, _ = jax.lax.fori_loop(1, K, body, (idxs0, dists0))
    return idxs


def _sample_farthest_points(xyz, K):
    idx = jax.vmap(lambda p: _fps_single(p, K))(jax.lax.stop_gradient(xyz))
    new_xyz = jax.vmap(lambda p, i: p[i])(xyz, idx)
    return new_xyz, idx


def _query_ball_point(radius, nsample, xyz, new_xyz):
    d2 = jnp.sum((new_xyz[:, :, None, :] - xyz[:, None, :, :]) ** 2, axis=-1)
    dist = jnp.sqrt(jnp.maximum(d2, 0.0))
    group_idx = jnp.argsort(dist, axis=-1)[:, :, :nsample]
    gd = jnp.take_along_axis(dist, group_idx, axis=-1)
    mask = gd > radius
    group_first = jnp.broadcast_to(group_idx[:, :, :1], group_idx.shape)
    return jnp.where(mask, group_first, group_idx)


def _index_points(points, idx):
    return jax.vmap(lambda p, i: p[i])(points, idx)


def _bn(x, gamma, beta, axes):
    mean = jnp.mean(x, axis=axes, keepdims=True)
    var = jnp.mean((x - mean) ** 2, axis=axes, keepdims=True)
    xh = (x - mean) / jnp.sqrt(var + 1e-5)
    shape = [1] * x.ndim
    shape[1] = -1
    return xh * gamma.reshape(shape) + beta.reshape(shape)


def _sa(p, npoint, radius, nsample, xyz, points):
    new_xyz, _ = _sample_farthest_points(xyz, npoint)
    gidx = _query_ball_point(radius, nsample, xyz, new_xyz)
    gxyz = _index_points(xyz, gidx) - new_xyz[:, :, None, :]
    if points is not None:
        gpts = _index_points(points, gidx)
        feat = jnp.concatenate([gxyz, gpts], axis=-1)
    else:
        feat = gxyz
    x = jnp.transpose(feat, (0, 3, 1, 2))
    for W, b, g, be in zip(p["W"], p["b"], p["gamma"], p["beta"]):
        x = jnp.einsum('bcsk,oc->bosk', x, W) + b[None, :, None, None]
        x = jax.nn.relu(_bn(x, g, be, (0, 2, 3)))
    x = jnp.max(x, axis=-1)
    return new_xyz, jnp.transpose(x, (0, 2, 1))


def _three_nn_interp(xyz_src, xyz_dst, feats_dst):
    def one(xs, xd, fd):
        d = jnp.sum((xs[:, None, :] - xd[None, :, :]) ** 2, axis=-1)
        idx = jnp.argsort(d, axis=-1)[:, :3]
        dk = jnp.take_along_axis(d, idx, axis=-1)
        w = 1.0 / (dk + 1e-8)
        w = w / jnp.sum(w, axis=1, keepdims=True)
        f3 = fd[:, idx]
        return jnp.sum(f3 * w[None], axis=-1)
    return jax.vmap(one)(xyz_src, xyz_dst, feats_dst)


def _fp(p, xyz1, xyz2, feat1, feat2):
    if xyz2.shape[1] == 1:
        interp = jnp.repeat(feat2, xyz1.shape[1], axis=2)
    else:
        interp = _three_nn_interp(xyz1, xyz2, feat2)
    nf = jnp.concatenate([feat1, interp], axis=1) if feat1 is not None else interp
    for W, b, g, be in zip(p["W"], p["b"], p["gamma"], p["beta"]):
        nf = jnp.einsum('bcn,oc->bon', nf, W) + b[None, :, None]
        nf = jax.nn.relu(_bn(nf, g, be, (0, 2)))
    return nf


def _identity_k(x_ref, o_ref):
    o_ref[...] = x_ref[...]


def kernel(xyz, params):
    l1_xyz, l1_pts = _sa(params["sa1"], 4096, 0.2, 32, xyz, None)
    l2_xyz, l2_pts = _sa(params["sa2"], 1024, 0.4, 64, l1_xyz, l1_pts)
    l3_xyz, l3_pts = _sa(params["sa3"], 512, 0.6, 128, l2_xyz, l2_pts)
    f1 = jnp.transpose(l1_pts, (0, 2, 1))
    f2 = jnp.transpose(l2_pts, (0, 2, 1))
    f3 = jnp.transpose(l3_pts, (0, 2, 1))
    f2u = _fp(params["fp3"], l2_xyz, l3_xyz, f2, f3)
    f1u = _fp(params["fp2"], l1_xyz, l2_xyz, f1, f2u)
    out = _fp(params["fp1"], xyz, l1_xyz, None, f1u)
    return pl.pallas_call(
        _identity_k,
        out_shape=jax.ShapeDtypeStruct(out.shape, out.dtype),
    )(out)


# speedup vs baseline: 1.9557x; 1.9550x over previous
import functools

import jax
import jax.numpy as jnp
import numpy as np
from jax.experimental import pallas as pl
from jax.experimental.pallas import tpu as pltpu

_INTERPRET = False
_BIGF = 1e30
_BIGI = 2**30


def _pc(*a, **k):
    return pl.pallas_call(*a, interpret=_INTERPRET, **k)


def _fps_body(K, xr_ref, idx_ref, nx_ref, ny_ref, nz_ref, dists_ref):
    b = pl.program_id(0)
    x = xr_ref[0, 0]
    y = xr_ref[0, 1]
    z = xr_ref[0, 2]
    R = x.shape[0]
    flat = (jax.lax.broadcasted_iota(jnp.int32, (R, 128), 0) * 128
            + jax.lax.broadcasted_iota(jnp.int32, (R, 128), 1))
    dists_ref[...] = jnp.full((R, 128), _BIGF, jnp.float32)
    sel0 = flat == 0
    lx0 = jnp.sum(jnp.where(sel0, x, 0.0))
    ly0 = jnp.sum(jnp.where(sel0, y, 0.0))
    lz0 = jnp.sum(jnp.where(sel0, z, 0.0))
    idx_ref[b, 0] = 0
    nx_ref[b, 0] = lx0
    ny_ref[b, 0] = ly0
    nz_ref[b, 0] = lz0

    def body(i, carry):
        lx, ly, lz = carry
        d = (x - lx) ** 2 + (y - ly) ** 2 + (z - lz) ** 2
        dm = jnp.minimum(dists_ref[...], d)
        dists_ref[...] = dm
        m = jnp.max(dm)
        sel = dm == m
        mi = jnp.min(jnp.where(sel, flat, _BIGI))
        sel2 = flat == mi
        nlx = jnp.sum(jnp.where(sel2, x, 0.0))
        nly = jnp.sum(jnp.where(sel2, y, 0.0))
        nlz = jnp.sum(jnp.where(sel2, z, 0.0))
        idx_ref[b, i] = mi
        nx_ref[b, i] = nlx
        ny_ref[b, i] = nly
        nz_ref[b, i] = nlz
        return (nlx, nly, nlz)

    jax.lax.fori_loop(1, K, body, (lx0, ly0, lz0))


def _fps(xr, K):
    B = xr.shape[0]
    R = xr.shape[2]
    idx, nx, ny, nz = _pc(
        functools.partial(_fps_body, K),
        grid=(B,),
        in_specs=[pl.BlockSpec((1, 3, R, 128), lambda b: (b, 0, 0, 0))],
        out_specs=[pl.BlockSpec(memory_space=pltpu.SMEM)] * 4,
        out_shape=[jax.ShapeDtypeStruct((B, K), jnp.int32)]
        + [jax.ShapeDtypeStruct((B, K), jnp.float32)] * 3,
        scratch_shapes=[pltpu.VMEM((R, 128), jnp.float32)],
    )(xr)
    return idx, jnp.stack([nx, ny, nz], axis=-1)


def _ballq_body(nsample, r2, xr_ref, c_ref, gidx_ref):
    x = xr_ref[0, 0][None]
    y = xr_ref[0, 1][None]
    z = xr_ref[0, 2][None]
    Ts = c_ref.shape[1]
    R = x.shape[1]
    cx = c_ref[0, :, 0][:, None, None]
    cy = c_ref[0, :, 1][:, None, None]
    cz = c_ref[0, :, 2][:, None, None]
    d = (x - cx) ** 2 + (y - cy) ** 2 + (z - cz) ** 2
    d = jnp.where(d <= r2, d, _BIGF)
    flat = (jax.lax.broadcasted_iota(jnp.int32, (Ts, R, 128), 1) * 128
            + jax.lax.broadcasted_iota(jnp.int32, (Ts, R, 128), 2))
    colio = jax.lax.broadcasted_iota(jnp.int32, (Ts, nsample), 1)

    def body(j, carry):
        dc, gacc, first = carry
        m = jnp.min(jnp.min(dc, axis=2), axis=1)
        sel = dc == m[:, None, None]
        fl = jnp.min(jnp.min(jnp.where(sel, flat, _BIGI), axis=2), axis=1)
        first = jnp.where(j == 0, fl, first)
        out_j = jnp.where(m < _BIGF * 0.5, fl, first)
        gacc = jnp.where(colio == j, out_j[:, None], gacc)
        dc = jnp.where(flat == fl[:, None, None], _BIGF, dc)
        return (dc, gacc, first)

    gacc0 = jnp.zeros((Ts, nsample), jnp.int32)
    first0 = jnp.zeros((Ts,), jnp.int32)
    _, gacc, _ = jax.lax.fori_loop(0, nsample, body, (d, gacc0, first0))
    gidx_ref[0] = gacc


def _ballq(xr, centers, radius, nsample, Ts=8):
    B, _, R, _ = xr.shape
    S = centers.shape[1]
    r2 = np.float32(radius) * np.float32(radius)
    return _pc(
        functools.partial(_ballq_body, nsample, r2),
        grid=(B, S // Ts),
        in_specs=[
            pl.BlockSpec((1, 3, R, 128), lambda b, s: (b, 0, 0, 0)),
            pl.BlockSpec((1, Ts, 3), lambda b, s: (b, s, 0)),
        ],
        out_specs=pl.BlockSpec((1, Ts, nsample), lambda b, s: (b, s, 0)),
        out_shape=jax.ShapeDtypeStruct((B, S, nsample), jnp.int32),
    )(xr, centers)


def _three_nn_body(xr2_ref, c_ref, idx_ref, w_ref):
    x = xr2_ref[0, 0][None]
    y = xr2_ref[0, 1][None]
    z = xr2_ref[0, 2][None]
    Ts = c_ref.shape[1]
    R = x.shape[1]
    cx = c_ref[0, :, 0][:, None, None]
    cy = c_ref[0, :, 1][:, None, None]
    cz = c_ref[0, :, 2][:, None, None]
    d = (x - cx) ** 2 + (y - cy) ** 2 + (z - cz) ** 2
    flat = (jax.lax.broadcasted_iota(jnp.int32, (Ts, R, 128), 1) * 128
            + jax.lax.broadcasted_iota(jnp.int32, (Ts, R, 128), 2))
    colio = jax.lax.broadcasted_iota(jnp.int32, (Ts, 3), 1)

    def body(j, carry):
        dc, iacc, wacc = carry
        m = jnp.min(jnp.min(dc, axis=2), axis=1)
        sel = dc == m[:, None, None]
        fl = jnp.min(jnp.min(jnp.where(sel, flat, _BIGI), axis=2), axis=1)
        iacc = jnp.where(colio == j, fl[:, None], iacc)
        wacc = jnp.where(colio == j, (1.0 / (m + 1e-8))[:, None], wacc)
        dc = jnp.where(flat == fl[:, None, None], _BIGF, dc)
        return (dc, iacc, wacc)

    iacc0 = jnp.zeros((Ts, 3), jnp.int32)
    wacc0 = jnp.zeros((Ts, 3), jnp.float32)
    _, iacc, wacc = jax.lax.fori_loop(0, 3, body, (d, iacc0, wacc0))
    idx_ref[0] = iacc
    w_ref[0] = wacc / jnp.sum(wacc, axis=1, keepdims=True)


def _three_nn(src_xyz, xr2, Ts=8):
    B, n1, _ = src_xyz.shape
    R = xr2.shape[2]
    return _pc(
        _three_nn_body,
        grid=(B, n1 // Ts),
        in_specs=[
            pl.BlockSpec((1, 3, R, 128), lambda b, s: (b, 0, 0, 0)),
            pl.BlockSpec((1, Ts, 3), lambda b, s: (b, s, 0)),
        ],
        out_specs=[
            pl.BlockSpec((1, Ts, 3), lambda b, s: (b, s, 0)),
            pl.BlockSpec((1, Ts, 3), lambda b, s: (b, s, 0)),
        ],
        out_shape=[
            jax.ShapeDtypeStruct((B, n1, 3), jnp.int32),
            jax.ShapeDtypeStruct((B, n1, 3), jnp.float32),
        ],
    )(xr2, src_xyz)


def _rowg_body(tab_ref, idx_ref, out_ref):
    CH = out_ref.shape[0]

    def body(r, _):
        i = idx_ref[r]
        out_ref[pl.ds(r, 1), :] = tab_ref[pl.ds(i, 1), :]
        return 0

    jax.lax.fori_loop(0, CH, body, 0, unroll=8)


def _gather_rows(idx_flat, table, CH=512):
    M = idx_flat.shape[0]
    V, D = table.shape
    CHx = min(CH, M)
    return _pc(
        _rowg_body,
        grid=(M // CHx,),
        in_specs=[
            pl.BlockSpec((V, D), lambda i: (0, 0)),
            pl.BlockSpec((CHx,), lambda i: (i,), memory_space=pltpu.SMEM),
        ],
        out_specs=pl.BlockSpec((CHx, D), lambda i: (i, 0)),
        out_shape=jax.ShapeDtypeStruct((M, D), jnp.float32),
    )(table, idx_flat)


def _mm_body(transform, x_ref, *refs):
    i = 0
    if transform:
        mu_ref, var_ref, g_ref, be_ref = refs[0], refs[1], refs[2], refs[3]
        i = 4
    w_ref, b_ref, z_ref, s1_ref = refs[i], refs[i + 1], refs[i + 2], refs[i + 3]
    x = x_ref[...]
    if transform:
        xh = (x - mu_ref[...]) / jnp.sqrt(var_ref[...] + 1e-5)
        x = jnp.maximum(xh * g_ref[...] + be_ref[...], 0.0)
    z = jax.lax.dot_general(x, w_ref[...], (((1,), (1,)), ((), ())),
                            preferred_element_type=jnp.float32)
    z = z + b_ref[...]
    z_ref[...] = z
    ps1 = jnp.sum(z, 0, keepdims=True)

    @pl.when(pl.program_id(0) == 0)
    def _():
        s1_ref[...] = ps1

    @pl.when(pl.program_id(0) > 0)
    def _():
        s1_ref[...] += ps1


def _mm(x, w, b, bn=None, TM=512):
    M, Ci = x.shape
    o = w.shape[0]
    TMx = min(TM, M)
    transform = bn is not None
    args = [x]
    in_specs = [pl.BlockSpec((TMx, Ci), lambda i: (i, 0))]
    if transform:
        args += [t.reshape(1, Ci) for t in bn]
        in_specs += [pl.BlockSpec((1, Ci), lambda i: (0, 0))] * 4
    args += [w, b.reshape(1, o)]
    in_specs += [pl.BlockSpec((o, Ci), lambda i: (0, 0)),
                 pl.BlockSpec((1, o), lambda i: (0, 0))]
    z, s1 = _pc(
        functools.partial(_mm_body, transform),
        grid=(M // TMx,),
        in_specs=in_specs,
        out_specs=[pl.BlockSpec((TMx, o), lambda i: (i, 0)),
                   pl.BlockSpec((1, o), lambda i: (0, 0))],
        out_shape=[jax.ShapeDtypeStruct((M, o), jnp.float32),
                   jax.ShapeDtypeStruct((1, o), jnp.float32)],
    )(*args)
    return z, s1


def _sumsq_body(z_ref, mu_ref, ss_ref):
    dz = z_ref[...] - mu_ref[...]
    ps = jnp.sum(dz * dz, 0, keepdims=True)

    @pl.when(pl.program_id(0) == 0)
    def _():
        ss_ref[...] = ps

    @pl.when(pl.program_id(0) > 0)
    def _():
        ss_ref[...] += ps


def _sumsq(z, mu, TM=1024):
    M, o = z.shape
    TMx = min(TM, M)
    return _pc(
        _sumsq_body,
        grid=(M // TMx,),
        in_specs=[pl.BlockSpec((TMx, o), lambda i: (i, 0)),
                  pl.BlockSpec((1, o), lambda i: (0, 0))],
        out_specs=pl.BlockSpec((1, o), lambda i: (0, 0)),
        out_shape=jax.ShapeDtypeStruct((1, o), jnp.float32),
    )(z, mu.reshape(1, o))


def _layer(x, W, b, bn_prev, rshape):
    M = 1
    for t in rshape:
        M *= t
    z, s1 = _mm(x, W, b, bn_prev)
    mu = s1.reshape(-1) / M
    var = _sumsq(z, mu).reshape(-1) / M
    return z, mu, var


def _featprep_body(gf_ref, c_ref, out_ref):
    out_ref[...] = gf_ref[...] - c_ref[...][:, None, :]


def _featprep(gf3, cpad):
    BS, K, Cp = gf3.shape
    Tc = _pick_tc(BS, K, Cp)
    return _pc(
        _featprep_body,
        grid=(BS // Tc,),
        in_specs=[pl.BlockSpec((Tc, K, Cp), lambda i: (i, 0, 0)),
                  pl.BlockSpec((Tc, Cp), lambda i: (i, 0))],
        out_specs=pl.BlockSpec((Tc, K, Cp), lambda i: (i, 0, 0)),
        out_shape=jax.ShapeDtypeStruct((BS, K, Cp), jnp.float32),
    )(gf3, cpad)


def _pick_tc(BS, K, o):
    for t in (64, 32, 16, 8):
        if BS % t == 0 and t * K * o * 4 <= (1 << 21):
            return t
    return 8


def _maxpool_body(z_ref, mu_ref, var_ref, g_ref, be_ref, out_ref):
    xh = (z_ref[...] - mu_ref[...][None]) / jnp.sqrt(var_ref[...][None] + 1e-5)
    a = jnp.maximum(xh * g_ref[...][None] + be_ref[...][None], 0.0)
    out_ref[...] = jnp.max(a, axis=1)


def _maxpool(z3, bn):
    BS, K, o = z3.shape
    Tc = _pick_tc(BS, K, o)
    return _pc(
        _maxpool_body,
        grid=(BS // Tc,),
        in_specs=[pl.BlockSpec((Tc, K, o), lambda i: (i, 0, 0))]
        + [pl.BlockSpec((1, o), lambda i: (0, 0))] * 4,
        out_specs=pl.BlockSpec((Tc, o), lambda i: (i, 0)),
        out_shape=jax.ShapeDtypeStruct((BS, o), jnp.float32),
    )(z3, *[t.reshape(1, o) for t in bn])


def _bnrelu_body(z_ref, mu_ref, var_ref, g_ref, be_ref, out_ref):
    xh = (z_ref[...] - mu_ref[...]) / jnp.sqrt(var_ref[...] + 1e-5)
    out_ref[...] = jnp.maximum(xh * g_ref[...] + be_ref[...], 0.0)


def _bnrelu(z, bn, TM=1024):
    M, o = z.shape
    TMx = min(TM, M)
    return _pc(
        _bnrelu_body,
        grid=(M // TMx,),
        in_specs=[pl.BlockSpec((TMx, o), lambda i: (i, 0))]
        + [pl.BlockSpec((1, o), lambda i: (0, 0))] * 4,
        out_specs=pl.BlockSpec((TMx, o), lambda i: (i, 0)),
        out_shape=jax.ShapeDtypeStruct((M, o), jnp.float32),
    )(z, *[t.reshape(1, o) for t in bn])


def _interp3_body(g0_ref, g1_ref, g2_ref, wf_ref, out_ref):
    wf = wf_ref[...]
    out_ref[...] = (g0_ref[...] * wf[:, 0:1] + g1_ref[...] * wf[:, 1:2]
                    + g2_ref[...] * wf[:, 2:3])


def _interp3(g0, g1, g2, wf, TM=512):
    M, o = g0.shape
    TMx = min(TM, M)
    return _pc(
        _interp3_body,
        grid=(M // TMx,),
        in_specs=[pl.BlockSpec((TMx, o), lambda i: (i, 0))] * 3
        + [pl.BlockSpec((TMx, 3), lambda i: (i, 0))],
        out_specs=pl.BlockSpec((TMx, o), lambda i: (i, 0)),
        out_shape=jax.ShapeDtypeStruct((M, o), jnp.float32),
    )(g0, g1, g2, wf)


def _sa_stage(p, xyz, pts, npoint, radius, nsample):
    B, N, _ = xyz.shape
    S, K = npoint, nsample
    M = B * S * K
    xr = xyz.transpose(0, 2, 1).reshape(B, 3, N // 128, 128)
    fps_idx, new_xyz = _fps(xr, S)
    gidx = _ballq(xr, new_xyz, radius, nsample)

    C = 3 if pts is None else 3 + pts.shape[-1]
    Cp = ((C + 15) // 16) * 16
    tab = xyz if pts is None else jnp.concatenate([xyz, pts], axis=-1)
    tabp = jnp.pad(tab, ((0, 0), (0, 0), (0, Cp - C))).reshape(B * N, Cp)
    boff = (jnp.arange(B, dtype=jnp.int32) * N)[:, None, None]
    gflat = (gidx + boff).reshape(-1)
    gf = _gather_rows(gflat, tabp)
    cpad = jnp.pad(new_xyz, ((0, 0), (0, 0), (0, Cp - 3))).reshape(B * S, Cp)
    feat = _featprep(gf.reshape(B * S, K, Cp), cpad)

    W1p = jnp.pad(p["W"][0], ((0, 0), (0, Cp - C)))
    rs = (B, S, K)
    z1, mu1, v1 = _layer(feat.reshape(M, Cp), W1p, p["b"][0], None, rs)
    bn1 = (mu1, v1, p["gamma"][0], p["beta"][0])
    z2, mu2, v2 = _layer(z1, p["W"][1], p["b"][1], bn1, rs)
    bn2 = (mu2, v2, p["gamma"][1], p["beta"][1])
    z3, mu3, v3 = _layer(z2, p["W"][2], p["b"][2], bn2, rs)
    bn3 = (mu3, v3, p["gamma"][2], p["beta"][2])
    o3 = p["W"][2].shape[0]
    out = _maxpool(z3.reshape(B * S, K, o3), bn3)
    return new_xyz, out.reshape(B, S, o3)


def _fp_stage(p, xyz1, xyz2, x1_rows, f_rows):
    B, n1, _ = xyz1.shape
    n2 = xyz2.shape[1]
    M = B * n1
    xr2 = xyz2.transpose(0, 2, 1).reshape(B, 3, n2 // 128, 128)
    idx3, w3 = _three_nn(xyz1, xr2)

    boff = (jnp.arange(B, dtype=jnp.int32) * n2)[:, None, None]
    giflat = (idx3 + boff).reshape(-1)
    gg = _gather_rows(giflat, f_rows)
    C2 = f_rows.shape[1]
    gg3 = gg.reshape(M, 3, C2)
    interp = _interp3(gg3[:, 0], gg3[:, 1], gg3[:, 2], w3.reshape(M, 3))

    x = interp if x1_rows is None else jnp.concatenate([x1_rows, interp], -1)
    rs = (B, n1)
    z1, mu1, v1 = _layer(x, p["W"][0], p["b"][0], None, rs)
    bn1 = (mu1, v1, p["gamma"][0], p["beta"][0])
    z2, mu2, v2 = _layer(z1, p["W"][1], p["b"][1], bn1, rs)
    bn2 = (mu2, v2, p["gamma"][1], p["beta"][1])
    return _bnrelu(z2, bn2)


def _forward(xyz, params, n1, n2, n3):
    B, N, _ = xyz.shape
    l1_xyz, l1_pts = _sa_stage(params["sa1"], xyz, None, n1, 0.2, 32)
    l2_xyz, l2_pts = _sa_stage(params["sa2"], l1_xyz, l1_pts, n2, 0.4, 64)
    l3_xyz, l3_pts = _sa_stage(params["sa3"], l2_xyz, l2_pts, n3, 0.6, 128)

    f1 = l1_pts.reshape(B * n1, 128)
    f2 = l2_pts.reshape(B * n2, 256)
    f3 = l3_pts.reshape(B * n3, 512)

    f2u = _fp_stage(params["fp3"], l2_xyz, l3_xyz, f2, f3)
    f1u = _fp_stage(params["fp2"], l1_xyz, l2_xyz, f1, f2u)
    out = _fp_stage(params["fp1"], xyz, l1_xyz, None, f1u)
    return out.reshape(B, N, 64).transpose(0, 2, 1)


def kernel(xyz, params):
    return _forward(xyz, params, 4096, 1024, 512)


# speedup vs baseline: 1.9570x; 1.0006x over previous
import functools

import jax
import jax.numpy as jnp
import numpy as np
from jax.experimental import pallas as pl
from jax.experimental.pallas import tpu as pltpu

_BIGF = 1e30
_BIGI = 2**30


def _pc(*a, **k):
    return pl.pallas_call(*a, **k)


def _fps_body(K, xr_ref, idx_ref, nx_ref, ny_ref, nz_ref, dists_ref):
    b = pl.program_id(0)
    x = xr_ref[0, 0]
    y = xr_ref[0, 1]
    z = xr_ref[0, 2]
    R = x.shape[0]
    flat = (jax.lax.broadcasted_iota(jnp.int32, (R, 128), 0) * 128
            + jax.lax.broadcasted_iota(jnp.int32, (R, 128), 1))
    dists_ref[...] = jnp.full((R, 128), _BIGF, jnp.float32)
    sel0 = flat == 0
    lx0 = jnp.sum(jnp.where(sel0, x, 0.0))
    ly0 = jnp.sum(jnp.where(sel0, y, 0.0))
    lz0 = jnp.sum(jnp.where(sel0, z, 0.0))
    idx_ref[b, 0] = 0
    nx_ref[b, 0] = lx0
    ny_ref[b, 0] = ly0
    nz_ref[b, 0] = lz0

    def body(i, carry):
        lx, ly, lz = carry
        d = (x - lx) ** 2 + (y - ly) ** 2 + (z - lz) ** 2
        dm = jnp.minimum(dists_ref[...], d)
        dists_ref[...] = dm
        m = jnp.max(dm)
        sel = dm == m
        mi = jnp.min(jnp.where(sel, flat, _BIGI))
        sel2 = flat == mi
        nlx = jnp.sum(jnp.where(sel2, x, 0.0))
        nly = jnp.sum(jnp.where(sel2, y, 0.0))
        nlz = jnp.sum(jnp.where(sel2, z, 0.0))
        idx_ref[b, i] = mi
        nx_ref[b, i] = nlx
        ny_ref[b, i] = nly
        nz_ref[b, i] = nlz
        return (nlx, nly, nlz)

    jax.lax.fori_loop(1, K, body, (lx0, ly0, lz0))


def _fps(xr, K):
    B = xr.shape[0]
    R = xr.shape[2]
    idx, nx, ny, nz = _pc(
        functools.partial(_fps_body, K),
        grid=(B,),
        in_specs=[pl.BlockSpec((1, 3, R, 128), lambda b: (b, 0, 0, 0))],
        out_specs=[pl.BlockSpec(memory_space=pltpu.SMEM)] * 4,
        out_shape=[jax.ShapeDtypeStruct((B, K), jnp.int32)]
        + [jax.ShapeDtypeStruct((B, K), jnp.float32)] * 3,
        scratch_shapes=[pltpu.VMEM((R, 128), jnp.float32)],
    )(xr)
    return idx, jnp.stack([nx, ny, nz], axis=-1)


def _ballq_body(nsample, r2, xr_ref, c_ref, gidx_ref):
    x = xr_ref[0, 0][None]
    y = xr_ref[0, 1][None]
    z = xr_ref[0, 2][None]
    Ts = c_ref.shape[1]
    R = x.shape[1]
    cx = c_ref[0, :, 0][:, None, None]
    cy = c_ref[0, :, 1][:, None, None]
    cz = c_ref[0, :, 2][:, None, None]
    d = (x - cx) ** 2 + (y - cy) ** 2 + (z - cz) ** 2
    d = jnp.where(d <= r2, d, _BIGF)
    flat = (jax.lax.broadcasted_iota(jnp.int32, (Ts, R, 128), 1) * 128
            + jax.lax.broadcasted_iota(jnp.int32, (Ts, R, 128), 2))
    colio = jax.lax.broadcasted_iota(jnp.int32, (Ts, nsample), 1)

    def body(j, carry):
        dc, gacc, first = carry
        m = jnp.min(jnp.min(dc, axis=2), axis=1)
        sel = dc == m[:, None, None]
        fl = jnp.min(jnp.min(jnp.where(sel, flat, _BIGI), axis=2), axis=1)
        first = jnp.where(j == 0, fl, first)
        out_j = jnp.where(m < _BIGF * 0.5, fl, first)
        gacc = jnp.where(colio == j, out_j[:, None], gacc)
        dc = jnp.where(flat == fl[:, None, None], _BIGF, dc)
        return (dc, gacc, first)

    gacc0 = jnp.zeros((Ts, nsample), jnp.int32)
    first0 = jnp.zeros((Ts,), jnp.int32)
    _, gacc, _ = jax.lax.fori_loop(0, nsample, body, (d, gacc0, first0))
    gidx_ref[0] = gacc


def _ballq(xr, centers, radius, nsample, Ts=8):
    B, _, R, _ = xr.shape
    S = centers.shape[1]
    r2 = np.float32(radius) * np.float32(radius)
    return _pc(
        functools.partial(_ballq_body, nsample, r2),
        grid=(B, S // Ts),
        in_specs=[
            pl.BlockSpec((1, 3, R, 128), lambda b, s: (b, 0, 0, 0)),
            pl.BlockSpec((1, Ts, 3), lambda b, s: (b, s, 0)),
        ],
        out_specs=pl.BlockSpec((1, Ts, nsample), lambda b, s: (b, s, 0)),
        out_shape=jax.ShapeDtypeStruct((B, S, nsample), jnp.int32),
    )(xr, centers)


def _three_nn_body(xr2_ref, c_ref, idx_ref, w_ref):
    x = xr2_ref[0, 0][None]
    y = xr2_ref[0, 1][None]
    z = xr2_ref[0, 2][None]
    Ts = c_ref.shape[1]
    R = x.shape[1]
    cx = c_ref[0, :, 0][:, None, None]
    cy = c_ref[0, :, 1][:, None, None]
    cz = c_ref[0, :, 2][:, None, None]
    d = (x - cx) ** 2 + (y - cy) ** 2 + (z - cz) ** 2
    flat = (jax.lax.broadcasted_iota(jnp.int32, (Ts, R, 128), 1) * 128
            + jax.lax.broadcasted_iota(jnp.int32, (Ts, R, 128), 2))
    colio = jax.lax.broadcasted_iota(jnp.int32, (Ts, 3), 1)

    def body(j, carry):
        dc, iacc, wacc = carry
        m = jnp.min(jnp.min(dc, axis=2), axis=1)
        sel = dc == m[:, None, None]
        fl = jnp.min(jnp.min(jnp.where(sel, flat, _BIGI), axis=2), axis=1)
        iacc = jnp.where(colio == j, fl[:, None], iacc)
        wacc = jnp.where(colio == j, (1.0 / (m + 1e-8))[:, None], wacc)
        dc = jnp.where(flat == fl[:, None, None], _BIGF, dc)
        return (dc, iacc, wacc)

    iacc0 = jnp.zeros((Ts, 3), jnp.int32)
    wacc0 = jnp.zeros((Ts, 3), jnp.float32)
    _, iacc, wacc = jax.lax.fori_loop(0, 3, body, (d, iacc0, wacc0))
    idx_ref[0] = iacc
    w_ref[0] = wacc / jnp.sum(wacc, axis=1, keepdims=True)


def _three_nn(src_xyz, xr2, Ts=8):
    B, n1, _ = src_xyz.shape
    R = xr2.shape[2]
    return _pc(
        _three_nn_body,
        grid=(B, n1 // Ts),
        in_specs=[
            pl.BlockSpec((1, 3, R, 128), lambda b, s: (b, 0, 0, 0)),
            pl.BlockSpec((1, Ts, 3), lambda b, s: (b, s, 0)),
        ],
        out_specs=[
            pl.BlockSpec((1, Ts, 3), lambda b, s: (b, s, 0)),
            pl.BlockSpec((1, Ts, 3), lambda b, s: (b, s, 0)),
        ],
        out_shape=[
            jax.ShapeDtypeStruct((B, n1, 3), jnp.int32),
            jax.ShapeDtypeStruct((B, n1, 3), jnp.float32),
        ],
    )(xr2, src_xyz)


def _rowg_body(tab_ref, idx_ref, out_ref):
    CH = out_ref.shape[0]

    def body(r, _):
        i = idx_ref[r]
        out_ref[pl.ds(r, 1), :] = tab_ref[pl.ds(i, 1), :]
        return 0

    jax.lax.fori_loop(0, CH, body, 0, unroll=8)


def _gather_rows(idx_flat, table, CH=512):
    M = idx_flat.shape[0]
    V, D = table.shape
    CHx = min(CH, M)
    return _pc(
        _rowg_body,
        grid=(M // CHx,),
        in_specs=[
            pl.BlockSpec((V, D), lambda i: (0, 0)),
            pl.BlockSpec((CHx,), lambda i: (i,), memory_space=pltpu.SMEM),
        ],
        out_specs=pl.BlockSpec((CHx, D), lambda i: (i, 0)),
        out_shape=jax.ShapeDtypeStruct((M, D), jnp.float32),
    )(table, idx_flat)


def _mm_body(transform, x_ref, *refs):
    i = 0
    if transform:
        mu_ref, var_ref, g_ref, be_ref = refs[0], refs[1], refs[2], refs[3]
        i = 4
    w_ref, b_ref, z_ref, s1_ref = refs[i], refs[i + 1], refs[i + 2], refs[i + 3]
    x = x_ref[...]
    if transform:
        xh = (x - mu_ref[...]) / jnp.sqrt(var_ref[...] + 1e-5)
        x = jnp.maximum(xh * g_ref[...] + be_ref[...], 0.0)
    z = jax.lax.dot_general(x, w_ref[...], (((1,), (1,)), ((), ())),
                            preferred_element_type=jnp.float32)
    z = z + b_ref[...]
    z_ref[...] = z
    ps1 = jnp.sum(z, 0, keepdims=True)

    @pl.when(pl.program_id(0) == 0)
    def _():
        s1_ref[...] = ps1

    @pl.when(pl.program_id(0) > 0)
    def _():
        s1_ref[...] += ps1


def _mm(x, w, b, bn=None, TM=512):
    M, Ci = x.shape
    o = w.shape[0]
    TMx = min(TM, M)
    transform = bn is not None
    args = [x]
    in_specs = [pl.BlockSpec((TMx, Ci), lambda i: (i, 0))]
    if transform:
        args += [t.reshape(1, Ci) for t in bn]
        in_specs += [pl.BlockSpec((1, Ci), lambda i: (0, 0))] * 4
    args += [w, b.reshape(1, o)]
    in_specs += [pl.BlockSpec((o, Ci), lambda i: (0, 0)),
                 pl.BlockSpec((1, o), lambda i: (0, 0))]
    z, s1 = _pc(
        functools.partial(_mm_body, transform),
        grid=(M // TMx,),
        in_specs=in_specs,
        out_specs=[pl.BlockSpec((TMx, o), lambda i: (i, 0)),
                   pl.BlockSpec((1, o), lambda i: (0, 0))],
        out_shape=[jax.ShapeDtypeStruct((M, o), jnp.float32),
                   jax.ShapeDtypeStruct((1, o), jnp.float32)],
    )(*args)
    return z, s1


def _sumsq_body(z_ref, mu_ref, ss_ref):
    dz = z_ref[...] - mu_ref[...]
    ps = jnp.sum(dz * dz, 0, keepdims=True)

    @pl.when(pl.program_id(0) == 0)
    def _():
        ss_ref[...] = ps

    @pl.when(pl.program_id(0) > 0)
    def _():
        ss_ref[...] += ps


def _sumsq(z, mu, TM=1024):
    M, o = z.shape
    TMx = min(TM, M)
    return _pc(
        _sumsq_body,
        grid=(M // TMx,),
        in_specs=[pl.BlockSpec((TMx, o), lambda i: (i, 0)),
                  pl.BlockSpec((1, o), lambda i: (0, 0))],
        out_specs=pl.BlockSpec((1, o), lambda i: (0, 0)),
        out_shape=jax.ShapeDtypeStruct((1, o), jnp.float32),
    )(z, mu.reshape(1, o))


def _layer(x, W, b, bn_prev, rshape):
    M = 1
    for t in rshape:
        M *= t
    z, s1 = _mm(x, W, b, bn_prev)
    mu = s1.reshape(-1) / M
    var = _sumsq(z, mu).reshape(-1) / M
    return z, mu, var


def _featprep_body(gf_ref, c_ref, out_ref):
    out_ref[...] = gf_ref[...] - c_ref[...][:, None, :]


def _featprep(gf3, cpad):
    BS, K, Cp = gf3.shape
    Tc = _pick_tc(BS, K, Cp)
    return _pc(
        _featprep_body,
        grid=(BS // Tc,),
        in_specs=[pl.BlockSpec((Tc, K, Cp), lambda i: (i, 0, 0)),
                  pl.BlockSpec((Tc, Cp), lambda i: (i, 0))],
        out_specs=pl.BlockSpec((Tc, K, Cp), lambda i: (i, 0, 0)),
        out_shape=jax.ShapeDtypeStruct((BS, K, Cp), jnp.float32),
    )(gf3, cpad)


def _pick_tc(BS, K, o):
    for t in (64, 32, 16, 8):
        if BS % t == 0 and t * K * o * 4 <= (1 << 21):
            return t
    return 8


def _maxpool_body(z_ref, mu_ref, var_ref, g_ref, be_ref, out_ref):
    xh = (z_ref[...] - mu_ref[...][None]) / jnp.sqrt(var_ref[...][None] + 1e-5)
    a = jnp.maximum(xh * g_ref[...][None] + be_ref[...][None], 0.0)
    out_ref[...] = jnp.max(a, axis=1)


def _maxpool(z3, bn):
    BS, K, o = z3.shape
    Tc = _pick_tc(BS, K, o)
    return _pc(
        _maxpool_body,
        grid=(BS // Tc,),
        in_specs=[pl.BlockSpec((Tc, K, o), lambda i: (i, 0, 0))]
        + [pl.BlockSpec((1, o), lambda i: (0, 0))] * 4,
        out_specs=pl.BlockSpec((Tc, o), lambda i: (i, 0)),
        out_shape=jax.ShapeDtypeStruct((BS, o), jnp.float32),
    )(z3, *[t.reshape(1, o) for t in bn])


def _bnrelu_body(z_ref, mu_ref, var_ref, g_ref, be_ref, out_ref):
    xh = (z_ref[...] - mu_ref[...]) / jnp.sqrt(var_ref[...] + 1e-5)
    out_ref[...] = jnp.maximum(xh * g_ref[...] + be_ref[...], 0.0)


def _bnrelu(z, bn, TM=1024):
    M, o = z.shape
    TMx = min(TM, M)
    return _pc(
        _bnrelu_body,
        grid=(M // TMx,),
        in_specs=[pl.BlockSpec((TMx, o), lambda i: (i, 0))]
        + [pl.BlockSpec((1, o), lambda i: (0, 0))] * 4,
        out_specs=pl.BlockSpec((TMx, o), lambda i: (i, 0)),
        out_shape=jax.ShapeDtypeStruct((M, o), jnp.float32),
    )(z, *[t.reshape(1, o) for t in bn])


def _interp3_body(g0_ref, g1_ref, g2_ref, wf_ref, out_ref):
    wf = wf_ref[...]
    out_ref[...] = (g0_ref[...] * wf[:, 0:1] + g1_ref[...] * wf[:, 1:2]
                    + g2_ref[...] * wf[:, 2:3])


def _interp3(g0, g1, g2, wf, TM=512):
    M, o = g0.shape
    TMx = min(TM, M)
    return _pc(
        _interp3_body,
        grid=(M // TMx,),
        in_specs=[pl.BlockSpec((TMx, o), lambda i: (i, 0))] * 3
        + [pl.BlockSpec((TMx, 3), lambda i: (i, 0))],
        out_specs=pl.BlockSpec((TMx, o), lambda i: (i, 0)),
        out_shape=jax.ShapeDtypeStruct((M, o), jnp.float32),
    )(g0, g1, g2, wf)


def _sa_stage(p, xyz, pts, npoint, radius, nsample):
    B, N, _ = xyz.shape
    S, K = npoint, nsample
    M = B * S * K
    xr = xyz.transpose(0, 2, 1).reshape(B, 3, N // 128, 128)
    fps_idx, new_xyz = _fps(xr, S)
    gidx = _ballq(xr, new_xyz, radius, nsample)

    C = 3 if pts is None else 3 + pts.shape[-1]
    Cp = ((C + 15) // 16) * 16
    tab = xyz if pts is None else jnp.concatenate([xyz, pts], axis=-1)
    tabp = jnp.pad(tab, ((0, 0), (0, 0), (0, Cp - C))).reshape(B * N, Cp)
    boff = (jnp.arange(B, dtype=jnp.int32) * N)[:, None, None]
    gflat = (gidx + boff).reshape(-1)
    gf = _gather_rows(gflat, tabp)
    cpad = jnp.pad(new_xyz, ((0, 0), (0, 0), (0, Cp - 3))).reshape(B * S, Cp)
    feat = _featprep(gf.reshape(B * S, K, Cp), cpad)

    W1p = jnp.pad(p["W"][0], ((0, 0), (0, Cp - C)))
    rs = (B, S, K)
    z1, mu1, v1 = _layer(feat.reshape(M, Cp), W1p, p["b"][0], None, rs)
    bn1 = (mu1, v1, p["gamma"][0], p["beta"][0])
    z2, mu2, v2 = _layer(z1, p["W"][1], p["b"][1], bn1, rs)
    bn2 = (mu2, v2, p["gamma"][1], p["beta"][1])
    z3, mu3, v3 = _layer(z2, p["W"][2], p["b"][2], bn2, rs)
    bn3 = (mu3, v3, p["gamma"][2], p["beta"][2])
    o3 = p["W"][2].shape[0]
    out = _maxpool(z3.reshape(B * S, K, o3), bn3)
    return new_xyz, out.reshape(B, S, o3)


def _fp_stage(p, xyz1, xyz2, x1_rows, f_rows):
    B, n1, _ = xyz1.shape
    n2 = xyz2.shape[1]
    M = B * n1
    xr2 = xyz2.transpose(0, 2, 1).reshape(B, 3, n2 // 128, 128)
    idx3, w3 = _three_nn(xyz1, xr2)

    boff = (jnp.arange(B, dtype=jnp.int32) * n2)[:, None, None]
    giflat = (idx3 + boff).reshape(-1)
    gg = _gather_rows(giflat, f_rows)
    C2 = f_rows.shape[1]
    gg3 = gg.reshape(M, 3, C2)
    interp = _interp3(gg3[:, 0], gg3[:, 1], gg3[:, 2], w3.reshape(M, 3))

    x = interp if x1_rows is None else jnp.concatenate([x1_rows, interp], -1)
    rs = (B, n1)
    z1, mu1, v1 = _layer(x, p["W"][0], p["b"][0], None, rs)
    bn1 = (mu1, v1, p["gamma"][0], p["beta"][0])
    z2, mu2, v2 = _layer(z1, p["W"][1], p["b"][1], bn1, rs)
    bn2 = (mu2, v2, p["gamma"][1], p["beta"][1])
    return _bnrelu(z2, bn2)


def _forward(xyz, params, n1, n2, n3):
    B, N, _ = xyz.shape
    l1_xyz, l1_pts = _sa_stage(params["sa1"], xyz, None, n1, 0.2, 32)
    l2_xyz, l2_pts = _sa_stage(params["sa2"], l1_xyz, l1_pts, n2, 0.4, 64)
    l3_xyz, l3_pts = _sa_stage(params["sa3"], l2_xyz, l2_pts, n3, 0.6, 128)

    f1 = l1_pts.reshape(B * n1, 128)
    f2 = l2_pts.reshape(B * n2, 256)
    f3 = l3_pts.reshape(B * n3, 512)

    f2u = _fp_stage(params["fp3"], l2_xyz, l3_xyz, f2, f3)
    f1u = _fp_stage(params["fp2"], l1_xyz, l2_xyz, f1, f2u)
    out = _fp_stage(params["fp1"], xyz, l1_xyz, None, f1u)
    return out.reshape(B, N, 64).transpose(0, 2, 1)


def kernel(xyz, params):
    return _forward(xyz, params, 4096, 1024, 512)
